# baseline (speedup 1.0000x reference)
import functools

import jax
import jax.numpy as jnp
from jax import lax
from jax.experimental import pallas as pl
from jax.experimental.pallas import tpu as pltpu
from jax.experimental.pallas import tpu_sc as plsc

_L = 16


def kernel(x, probe_x, probe_y):
    B, H, W = x.shape
    N = probe_x.shape[0]
    assert (B * N) % (32 * _L) == 0
    table = x.reshape(-1)

    mesh = plsc.VectorSubcoreMesh(core_axis_name="c", subcore_axis_name="s")
    chunks_per_batch = N // _L

    @functools.partial(
        pl.kernel,
        mesh=mesh,
        out_type=jax.ShapeDtypeStruct((B * N,), jnp.float32),
        scratch_types=[
            pltpu.VMEM((_L,), jnp.int32),
            pltpu.VMEM((_L,), jnp.int32),
            pltpu.VMEM((_L,), jnp.float32),
            pltpu.SemaphoreType.DMA,
        ],
    )
    def gather_kernel(table_hbm, px_hbm, py_hbm, out_hbm,
                      px_v, py_v, vals_v, sem):
        wid = lax.axis_index("s") * 2 + lax.axis_index("c")
        b = wid // chunks_per_batch
        c = wid % chunks_per_batch
        pltpu.sync_copy(px_hbm.at[pl.ds(c * _L, _L)], px_v)
        pltpu.sync_copy(py_hbm.at[pl.ds(c * _L, _L)], py_v)
        addr = b * (H * W) + py_v[...] * W + px_v[...]
        pltpu.async_copy(table_hbm.at[addr], vals_v, sem).wait()
        pltpu.sync_copy(vals_v, out_hbm.at[pl.ds(wid * _L, _L)])

    out = gather_kernel(table, probe_x, probe_y)
    return out.reshape(B, N)


# speedup vs baseline: 4.2138x; 4.2138x over previous
import functools

import jax
import jax.numpy as jnp
from jax import lax
from jax.experimental import pallas as pl
from jax.experimental.pallas import tpu as pltpu
from jax.experimental.pallas import tpu_sc as plsc

_L = 16


def kernel(x, probe_x, probe_y):
    B, H, W = x.shape
    N = probe_x.shape[0]
    assert (B * N) % (32 * _L) == 0

    table = x.reshape(B * H, W)

    mesh = plsc.VectorSubcoreMesh(core_axis_name="c", subcore_axis_name="s")
    chunks_per_batch = N // _L

    @functools.partial(
        pl.kernel,
        mesh=mesh,
        out_type=jax.ShapeDtypeStruct((B, N), jnp.float32),
        scratch_types=[
            pltpu.VMEM((_L,), jnp.int32),
            pltpu.VMEM((_L,), jnp.int32),
            pltpu.VMEM((_L, 2048), jnp.float32),
            pltpu.VMEM((_L,), jnp.float32),
            pltpu.VMEM_SHARED((16 * _L * 2048,), jnp.float32),
            pltpu.SemaphoreType.DMA,
        ],
    )
    def gather_kernel(table_hbm, px_hbm, py_hbm, out_hbm,
                      px_v, py_v, rows_v, vals_v, shared_v, sem):
        sid = lax.axis_index("s")
        wid = sid * 2 + lax.axis_index("c")
        b = wid // chunks_per_batch
        c = wid % chunks_per_batch
        pltpu.sync_copy(px_hbm.at[pl.ds(c * _L, _L)], px_v)
        pltpu.sync_copy(py_hbm.at[pl.ds(c * _L, _L)], py_v)
        row = b * H + py_v[...]
        pltpu.async_copy(table_hbm.at[row], rows_v, sem).wait()
        base = sid * (_L * W)
        for t in range(_L):
            pltpu.sync_copy(rows_v.at[t], shared_v.at[pl.ds(base + t * W, W)])
        flat_idx = base + lax.iota(jnp.int32, _L) * W + px_v[...]
        pltpu.async_copy(shared_v.at[flat_idx], vals_v, sem).wait()
        pltpu.sync_copy(vals_v, out_hbm.at[b, pl.ds(c * _L, _L)])

    return gather_kernel(table, probe_x, probe_y)


# speedup vs baseline: 5.1116x; 1.2131x over previous
import functools

import jax
import jax.numpy as jnp
from jax import lax
from jax.experimental import pallas as pl
from jax.experimental.pallas import tpu as pltpu
from jax.experimental.pallas import tpu_sc as plsc

_L = 16
_TW = 128


def kernel(x, probe_x, probe_y):
    B, H, W = x.shape
    N = probe_x.shape[0]
    assert (B * N) % (32 * _L) == 0

    table = x.reshape(B * H, W)

    mesh = plsc.VectorSubcoreMesh(core_axis_name="c", subcore_axis_name="s")
    chunks_per_batch = N // _L

    @functools.partial(
        pl.kernel,
        mesh=mesh,
        out_type=jax.ShapeDtypeStruct((B, N), jnp.float32),
        scratch_types=[
            pltpu.VMEM((_L,), jnp.int32),
            pltpu.VMEM((_L,), jnp.int32),
            pltpu.VMEM((_L, 2048), jnp.float32),
            pltpu.VMEM((_L,), jnp.float32),
            pltpu.VMEM_SHARED((16 * _L * _TW,), jnp.float32),
            pltpu.SemaphoreType.DMA,
            pltpu.SemaphoreType.DMA,
        ],
    )
    def gather_kernel(table_hbm, px_hbm, py_hbm, out_hbm,
                      px_v, py_v, rows_v, vals_v, shared_v,
                      sem, sem2):
        sid = lax.axis_index("s")
        wid = sid * 2 + lax.axis_index("c")
        b = wid // chunks_per_batch
        c = wid % chunks_per_batch
        pltpu.sync_copy(px_hbm.at[pl.ds(c * _L, _L)], px_v)
        pltpu.sync_copy(py_hbm.at[pl.ds(c * _L, _L)], py_v)
        row = b * H + py_v[...]
        pltpu.async_copy(table_hbm.at[row], rows_v, sem).wait()
        base = sid * (_L * _TW)
        x0_vec = lax.bitwise_and(px_v[...], ~(_TW - 1))
        copies = []
        for t in range(_L):
            x0_t = pl.multiple_of(x0_vec[t], _TW)
            copies.append(pltpu.make_async_copy(
                rows_v.at[t, pl.ds(x0_t, _TW)],
                shared_v.at[pl.ds(base + t * _TW, _TW)], sem2))
        for cp in copies:
            cp.start()
        for cp in copies:
            cp.wait()
        flat_idx = (base + lax.iota(jnp.int32, _L) * _TW
                    + lax.bitwise_and(px_v[...], _TW - 1))
        pltpu.async_copy(shared_v.at[flat_idx], vals_v, sem).wait()
        pltpu.sync_copy(vals_v, out_hbm.at[b, pl.ds(c * _L, _L)])

    return gather_kernel(table, probe_x, probe_y)
